# Initial kernel scaffold; baseline (speedup 1.0000x reference)
import jax
import jax.numpy as jnp
from jax.experimental import pallas as pl


def kernel(x, W1, W2):
    raise NotImplementedError("write your pallas kernel here")


# speedup vs baseline: 1.1795x; 1.1795x over previous
import functools

import jax
import jax.numpy as jnp
from jax.experimental import pallas as pl
from jax.experimental.pallas import tpu as pltpu

D = 2048
SK = 8
NSCORE = SK * SK
TOP_K = 8
BLK = 1024


def _router_kernel(x_ref, w_ref, idx_ref, probs_ref, scores_ref):
    x = x_ref[...]
    w = w_ref[...]
    s = jax.lax.dot_general(
        x, w, (((1,), (1,)), ((), ())),
        preferred_element_type=jnp.float32)
    s1 = s[:, :SK]
    s2 = s[:, SK:]
    scores = (s1[:, :, None] + s2[:, None, :]).reshape(BLK, NSCORE)
    scores_ref[...] = scores

    iota = jax.lax.broadcasted_iota(jnp.int32, (BLK, NSCORE), 1)
    cur = scores
    neg_inf = jnp.float32(-jnp.inf)
    for k in range(TOP_K):
        m = jnp.max(cur, axis=-1, keepdims=True)
        is_max = cur == m
        idx = jnp.min(jnp.where(is_max, iota, NSCORE), axis=-1,
                      keepdims=True)
        idx_ref[:, k:k + 1] = idx
        probs_ref[:, k:k + 1] = m
        cur = jnp.where(iota == idx, neg_inf, cur)

    v = probs_ref[...]
    e = jnp.exp(v - v[:, :1])
    probs_ref[...] = e / jnp.sum(e, axis=-1, keepdims=True)


@jax.jit
def kernel(x, W1, W2):
    n_tok = x.shape[0]
    w = jnp.concatenate([W1, W2], axis=0)
    grid = (n_tok // BLK,)
    out = pl.pallas_call(
        _router_kernel,
        grid=grid,
        in_specs=[
            pl.BlockSpec((BLK, D), lambda i: (i, 0)),
            pl.BlockSpec((2 * SK, D), lambda i: (0, 0)),
        ],
        out_specs=[
            pl.BlockSpec((BLK, TOP_K), lambda i: (i, 0)),
            pl.BlockSpec((BLK, TOP_K), lambda i: (i, 0)),
            pl.BlockSpec((BLK, NSCORE), lambda i: (i, 0)),
        ],
        out_shape=[
            jax.ShapeDtypeStruct((n_tok, TOP_K), jnp.int32),
            jax.ShapeDtypeStruct((n_tok, TOP_K), jnp.float32),
            jax.ShapeDtypeStruct((n_tok, NSCORE), jnp.float32),
        ],
    )(x, w)
    return (out[0], out[1], out[2])


# speedup vs baseline: 1.2936x; 1.0968x over previous
import jax
import jax.numpy as jnp
from jax.experimental import pallas as pl

D = 2048
SK = 8
NSCORE = SK * SK
TOP_K = 8
BLK = 1024

_SIGN_FIX = 0x7FFFFFFF
_LOW_MASK = ~63
_NEG_INF_KEY = -(2 ** 31)


def _router_kernel(x_ref, w_ref, idx_ref, probs_ref, scores_ref):
    x = x_ref[...]
    w = w_ref[...]
    s = jax.lax.dot_general(
        x, w, (((1,), (1,)), ((), ())),
        preferred_element_type=jnp.float32)
    scores = s[:, :NSCORE] + s[:, NSCORE:]
    scores_ref[...] = scores

    bits = jax.lax.bitcast_convert_type(scores, jnp.int32)
    skey = jnp.where(bits >= 0, bits, bits ^ _SIGN_FIX)
    iota = jax.lax.broadcasted_iota(jnp.int32, (BLK, NSCORE), 1)
    key = (skey & _LOW_MASK) | (NSCORE - 1 - iota)

    for k in range(TOP_K):
        kmax = jnp.max(key, axis=-1, keepdims=True)
        idx_ref[:, k:k + 1] = (NSCORE - 1) - (kmax & (NSCORE - 1))
        vb = kmax & _LOW_MASK
        fb = jnp.where(vb >= 0, vb, vb ^ _SIGN_FIX)
        probs_ref[:, k:k + 1] = jax.lax.bitcast_convert_type(fb, jnp.float32)
        key = jnp.where(key == kmax, _NEG_INF_KEY, key)

    v = probs_ref[...]
    e = jnp.exp(v - v[:, :1])
    probs_ref[...] = e / jnp.sum(e, axis=-1, keepdims=True)


@jax.jit
def kernel(x, W1, W2):
    n_tok = x.shape[0]
    wc = jnp.concatenate(
        [jnp.repeat(W1, SK, axis=0), jnp.tile(W2, (SK, 1))], axis=0)
    grid = (n_tok // BLK,)
    out = pl.pallas_call(
        _router_kernel,
        grid=grid,
        in_specs=[
            pl.BlockSpec((BLK, D), lambda i: (i, 0)),
            pl.BlockSpec((2 * NSCORE, D), lambda i: (0, 0)),
        ],
        out_specs=[
            pl.BlockSpec((BLK, TOP_K), lambda i: (i, 0)),
            pl.BlockSpec((BLK, TOP_K), lambda i: (i, 0)),
            pl.BlockSpec((BLK, NSCORE), lambda i: (i, 0)),
        ],
        out_shape=[
            jax.ShapeDtypeStruct((n_tok, TOP_K), jnp.int32),
            jax.ShapeDtypeStruct((n_tok, TOP_K), jnp.float32),
            jax.ShapeDtypeStruct((n_tok, NSCORE), jnp.float32),
        ],
    )(x, wc)
    return (out[0], out[1], out[2])
